# baseline (speedup 1.0000x reference)
import functools

import jax
import jax.numpy as jnp
from jax import lax
from jax.experimental import pallas as pl
from jax.experimental.pallas import tpu as pltpu
from jax.experimental.pallas import tpu_sc as plsc

BIN = 100
PAD = 128
HID = 128
TB = 2048


def _idx_body(x_ref, w1_ref, b1_ref, w2_ref, b2_ref, idx_ref):
    xb = x_ref[...]
    h1 = jnp.maximum(xb * w1_ref[...] + b1_ref[...], 0.0)
    h2 = jax.lax.dot_general(
        h1, w2_ref[...], (((1,), (0,)), ((), ())),
        precision=jax.lax.Precision.DEFAULT,
        preferred_element_type=jnp.float32,
    ) + b2_ref[...]
    h2 = jnp.maximum(h2, 0.0)
    m = jnp.max(h2, axis=1, keepdims=True)
    lane = jax.lax.broadcasted_iota(jnp.int32, (TB, PAD), 1)
    idx = jnp.min(jnp.where(h2 >= m, lane, PAD), axis=1, keepdims=True)
    idx_ref[...] = idx


def _compute_idx(xf, W1, b1, W2, b2):
    N = xf.shape[0]
    w1p = jnp.zeros((1, PAD), jnp.float32).at[:, :BIN].set(W1)
    b1p = jnp.zeros((1, PAD), jnp.float32).at[:, :BIN].set(b1)
    w2p = jnp.zeros((PAD, PAD), jnp.float32).at[:BIN, :BIN].set(W2)
    b2p = jnp.zeros((1, PAD), jnp.float32).at[:, :BIN].set(b2)
    grid = N // TB
    idx = pl.pallas_call(
        _idx_body,
        grid=(grid,),
        in_specs=[
            pl.BlockSpec((TB, 1), lambda i: (i, 0)),
            pl.BlockSpec((1, PAD), lambda i: (0, 0)),
            pl.BlockSpec((1, PAD), lambda i: (0, 0)),
            pl.BlockSpec((PAD, PAD), lambda i: (0, 0)),
            pl.BlockSpec((1, PAD), lambda i: (0, 0)),
        ],
        out_specs=pl.BlockSpec((TB, 1), lambda i: (i, 0)),
        out_shape=jax.ShapeDtypeStruct((N, 1), jnp.int32),
    )(xf, w1p, b1p, w2p, b2p)
    return idx.reshape(N)


_INFO = plsc.get_sparse_core_info()
_NC, _NS = _INFO.num_cores, _INFO.num_subcores
_NW = _NC * _NS
_CHUNK = 128


def _make_sc_gather(N):
    b_per_w = N // _NW
    n_chunks = b_per_w // _CHUNK
    mesh = plsc.VectorSubcoreMesh(core_axis_name="c", subcore_axis_name="s")

    @functools.partial(
        pl.kernel, mesh=mesh,
        out_type=jax.ShapeDtypeStruct((N, HID), jnp.float32),
        scratch_types=[
            pltpu.VMEM((2, _CHUNK), jnp.int32),
            pltpu.VMEM((2, _CHUNK, HID), jnp.float32),
            pltpu.SemaphoreType.DMA,
            pltpu.SemaphoreType.DMA,
        ],
    )
    def sc_gather(emb_hbm, idx_hbm, out_hbm, idx_v, rows_v, gsem, ssem):
        wid = lax.axis_index("s") * _NC + lax.axis_index("c")
        base = wid * b_per_w

        def chunk(i, slot):
            off = base + i * _CHUNK
            pltpu.sync_copy(idx_hbm.at[pl.ds(off, _CHUNK)], idx_v.at[slot])
            pltpu.async_copy(emb_hbm.at[idx_v.at[slot]], rows_v.at[slot],
                             gsem).wait()
            pltpu.async_copy(rows_v.at[slot], out_hbm.at[pl.ds(off, _CHUNK)],
                             ssem).wait()

        lax.fori_loop(0, n_chunks, lambda i, c: (chunk(i, 0), c)[1], 0)

    return sc_gather


def kernel(x, W1, b1, W2, b2, emb):
    B, L, _ = x.shape
    N = B * L
    xf = x.reshape(N, 1)
    idx = _compute_idx(xf, W1, b1, W2, b2)
    out = _make_sc_gather(N)(emb, idx)
    return out.reshape(B, L, HID)


# speedup vs baseline: 3.5474x; 3.5474x over previous
import functools

import jax
import jax.numpy as jnp
from jax import lax
from jax.experimental import pallas as pl
from jax.experimental.pallas import tpu as pltpu
from jax.experimental.pallas import tpu_sc as plsc

BIN = 100
PAD = 128
HID = 128
TB = 2048


def _idx_body(x_ref, w1_ref, b1_ref, w2_ref, b2_ref, idx_ref):
    xb = x_ref[...]
    h1 = jnp.maximum(xb * w1_ref[...] + b1_ref[...], 0.0)
    h2 = jax.lax.dot_general(
        h1, w2_ref[...], (((1,), (0,)), ((), ())),
        precision=jax.lax.Precision.DEFAULT,
        preferred_element_type=jnp.float32,
    ) + b2_ref[...]
    h2 = jnp.maximum(h2, 0.0)
    m = jnp.max(h2, axis=1, keepdims=True)
    lane = jax.lax.broadcasted_iota(jnp.int32, (TB, PAD), 1)
    idx = jnp.min(jnp.where(h2 >= m, lane, PAD), axis=1, keepdims=True)
    idx_ref[...] = idx


def _compute_idx(xf, W1, b1, W2, b2):
    N = xf.shape[0]
    w1p = jnp.zeros((1, PAD), jnp.float32).at[:, :BIN].set(W1)
    b1p = jnp.zeros((1, PAD), jnp.float32).at[:, :BIN].set(b1)
    w2p = jnp.zeros((PAD, PAD), jnp.float32).at[:BIN, :BIN].set(W2)
    b2p = jnp.zeros((1, PAD), jnp.float32).at[:, :BIN].set(b2)
    grid = N // TB
    idx = pl.pallas_call(
        _idx_body,
        grid=(grid,),
        in_specs=[
            pl.BlockSpec((TB, 1), lambda i: (i, 0)),
            pl.BlockSpec((1, PAD), lambda i: (0, 0)),
            pl.BlockSpec((1, PAD), lambda i: (0, 0)),
            pl.BlockSpec((PAD, PAD), lambda i: (0, 0)),
            pl.BlockSpec((1, PAD), lambda i: (0, 0)),
        ],
        out_specs=pl.BlockSpec((TB, 1), lambda i: (i, 0)),
        out_shape=jax.ShapeDtypeStruct((N, 1), jnp.int32),
    )(xf, w1p, b1p, w2p, b2p)
    return idx.reshape(N)


_INFO = plsc.get_sparse_core_info()
_NC, _NS = _INFO.num_cores, _INFO.num_subcores
_NW = _NC * _NS
_CHUNK = 256
_L = 16


def _make_sc_gather(N):
    b_per_w = N // _NW
    n_pairs = b_per_w // (2 * _CHUNK)
    n_groups = _CHUNK // _L
    mesh = plsc.VectorSubcoreMesh(core_axis_name="c", subcore_axis_name="s")

    @functools.partial(
        pl.kernel, mesh=mesh,
        out_type=jax.ShapeDtypeStruct((N, HID), jnp.float32),
        scratch_types=[
            pltpu.VMEM((BIN, HID), jnp.float32),
            pltpu.VMEM((_CHUNK,), jnp.int32),
            pltpu.VMEM((_CHUNK,), jnp.int32),
            pltpu.VMEM((_CHUNK, HID), jnp.float32),
            pltpu.VMEM((_CHUNK, HID), jnp.float32),
            pltpu.SemaphoreType.DMA,
            pltpu.SemaphoreType.DMA,
        ],
        compiler_params=pltpu.CompilerParams(needs_layout_passes=False),
    )
    def sc_gather(emb_hbm, idx_hbm, out_hbm, emb_v, idx0_v, idx1_v,
                  rows0_v, rows1_v, ssem0, ssem1):
        wid = lax.axis_index("s") * _NC + lax.axis_index("c")
        base = wid * b_per_w
        pltpu.sync_copy(emb_hbm, emb_v)
        lane = lax.iota(jnp.int32, _L)

        def fill(idx_v, rows_v, off):
            pltpu.sync_copy(idx_hbm.at[pl.ds(off, _CHUNK)], idx_v)

            def group(g, carry):
                idx16 = idx_v[pl.ds(g * _L, _L)]
                tok16 = lane + g * _L

                def cols(cc, carry2):
                    c0 = cc * 8
                    for u in range(8):
                        c_splat = jnp.full((_L,), c0 + u, jnp.int32)
                        v = plsc.load_gather(emb_v, [idx16, c_splat])
                        plsc.store_scatter(rows_v, [tok16, c_splat], v)
                    return carry2

                lax.fori_loop(0, HID // 8, cols, 0)
                return carry

            lax.fori_loop(0, n_groups, group, 0)

        def pair(p, carry):
            off0 = base + (2 * p) * _CHUNK
            off1 = off0 + _CHUNK
            fill(idx0_v, rows0_v, off0)
            pltpu.async_copy(rows0_v, out_hbm.at[pl.ds(off0, _CHUNK)],
                             ssem0).wait()
            fill(idx1_v, rows1_v, off1)
            pltpu.async_copy(rows1_v, out_hbm.at[pl.ds(off1, _CHUNK)],
                             ssem1).wait()
            return carry

        lax.fori_loop(0, n_pairs, pair, 0)

    return sc_gather


def kernel(x, W1, b1, W2, b2, emb):
    B, L, _ = x.shape
    N = B * L
    xf = x.reshape(N, 1)
    idx = _compute_idx(xf, W1, b1, W2, b2)
    out = _make_sc_gather(N)(emb, idx)
    return out.reshape(B, L, HID)


# speedup vs baseline: 4.9740x; 1.4021x over previous
import functools

import jax
import jax.numpy as jnp
from jax import lax
from jax.experimental import pallas as pl
from jax.experimental.pallas import tpu as pltpu
from jax.experimental.pallas import tpu_sc as plsc

BIN = 100
PAD = 128
HID = 128
TB = 2048


def _idx_body(x_ref, w1_ref, b1_ref, w2_ref, b2_ref, idx_ref):
    xb = x_ref[...]
    h1 = jnp.maximum(xb * w1_ref[...] + b1_ref[...], 0.0)
    h2 = jax.lax.dot_general(
        h1, w2_ref[...], (((1,), (0,)), ((), ())),
        precision=jax.lax.Precision.DEFAULT,
        preferred_element_type=jnp.float32,
    ) + b2_ref[...]
    h2 = jnp.maximum(h2, 0.0)
    m = jnp.max(h2, axis=1, keepdims=True)
    lane = jax.lax.broadcasted_iota(jnp.int32, (TB, PAD), 1)
    idx = jnp.min(jnp.where(h2 >= m, lane, PAD), axis=1, keepdims=True)
    idx_ref[...] = idx


def _compute_idx(xf, W1, b1, W2, b2):
    N = xf.shape[0]
    w1p = jnp.zeros((1, PAD), jnp.float32).at[:, :BIN].set(W1)
    b1p = jnp.zeros((1, PAD), jnp.float32).at[:, :BIN].set(b1)
    w2p = jnp.zeros((PAD, PAD), jnp.float32).at[:BIN, :BIN].set(W2)
    b2p = jnp.zeros((1, PAD), jnp.float32).at[:, :BIN].set(b2)
    grid = N // TB
    idx = pl.pallas_call(
        _idx_body,
        grid=(grid,),
        in_specs=[
            pl.BlockSpec((TB, 1), lambda i: (i, 0)),
            pl.BlockSpec((1, PAD), lambda i: (0, 0)),
            pl.BlockSpec((1, PAD), lambda i: (0, 0)),
            pl.BlockSpec((PAD, PAD), lambda i: (0, 0)),
            pl.BlockSpec((1, PAD), lambda i: (0, 0)),
        ],
        out_specs=pl.BlockSpec((TB, 1), lambda i: (i, 0)),
        out_shape=jax.ShapeDtypeStruct((N, 1), jnp.int32),
    )(xf, w1p, b1p, w2p, b2p)
    return idx.reshape(N)


_INFO = plsc.get_sparse_core_info()
_NC, _NS = _INFO.num_cores, _INFO.num_subcores
_NW = _NC * _NS
_CHUNK = 256
_L = 16


def _make_sc_gather(N):
    b_per_w = N // _NW
    n_pairs = b_per_w // (2 * _CHUNK)
    n_groups = _CHUNK // _L
    mesh = plsc.VectorSubcoreMesh(core_axis_name="c", subcore_axis_name="s")

    @functools.partial(
        pl.kernel, mesh=mesh,
        out_type=jax.ShapeDtypeStruct((N, HID), jnp.float32),
        scratch_types=[
            pltpu.VMEM((BIN, HID), jnp.float32),
            pltpu.VMEM((_CHUNK,), jnp.int32),
            pltpu.VMEM((_CHUNK,), jnp.int32),
            pltpu.VMEM((_CHUNK, HID), jnp.float32),
            pltpu.VMEM((_CHUNK, HID), jnp.float32),
            pltpu.SemaphoreType.DMA,
            pltpu.SemaphoreType.DMA,
        ],
        compiler_params=pltpu.CompilerParams(needs_layout_passes=False),
    )
    def sc_gather(emb_hbm, idx_hbm, out_hbm, emb_v, idx0_v, idx1_v,
                  rows0_v, rows1_v, ssem0, ssem1):
        wid = lax.axis_index("s") * _NC + lax.axis_index("c")
        base = wid * b_per_w
        pltpu.sync_copy(emb_hbm, emb_v)
        lane = lax.iota(jnp.int32, _L)

        def fill(idx_v, rows_v, off):
            pltpu.sync_copy(idx_hbm.at[pl.ds(off, _CHUNK)], idx_v)

            @plsc.parallel_loop(0, n_groups, unroll=2)
            def group(g):
                idx16 = idx_v[pl.ds(g * _L, _L)]
                tok16 = lane + g * _L

                @plsc.parallel_loop(0, HID, step=8, unroll=4)
                def cols(c0):
                    vs = []
                    for u in range(8):
                        c_splat = jnp.full((_L,), c0 + u, jnp.int32)
                        vs.append(plsc.load_gather(emb_v, [idx16, c_splat]))
                    for u in range(8):
                        c_splat = jnp.full((_L,), c0 + u, jnp.int32)
                        plsc.store_scatter(rows_v, [tok16, c_splat], vs[u])

        def pair(p, carry):
            off0 = base + (2 * p) * _CHUNK
            off1 = off0 + _CHUNK
            fill(idx0_v, rows0_v, off0)
            pltpu.async_copy(rows0_v, out_hbm.at[pl.ds(off0, _CHUNK)],
                             ssem0).wait()
            fill(idx1_v, rows1_v, off1)
            pltpu.async_copy(rows1_v, out_hbm.at[pl.ds(off1, _CHUNK)],
                             ssem1).wait()
            return carry

        lax.fori_loop(0, n_pairs, pair, 0)

    return sc_gather


def kernel(x, W1, b1, W2, b2, emb):
    B, L, _ = x.shape
    N = B * L
    xf = x.reshape(N, 1)
    idx = _compute_idx(xf, W1, b1, W2, b2)
    out = _make_sc_gather(N)(emb, idx)
    return out.reshape(B, L, HID)


# speedup vs baseline: 12.8505x; 2.5835x over previous
import functools

import jax
import jax.numpy as jnp
from jax import lax
from jax.experimental import pallas as pl
from jax.experimental.pallas import tpu as pltpu
from jax.experimental.pallas import tpu_sc as plsc

BIN = 100
PAD = 128
HID = 128
TB = 2048


def _idx_body(x_ref, w1_ref, b1_ref, w2_ref, b2_ref, idx_ref):
    xb = x_ref[...]
    h1 = jnp.maximum(xb * w1_ref[...] + b1_ref[...], 0.0)
    h2 = jax.lax.dot_general(
        h1, w2_ref[...], (((1,), (0,)), ((), ())),
        precision=jax.lax.Precision.DEFAULT,
        preferred_element_type=jnp.float32,
    ) + b2_ref[...]
    h2 = jnp.maximum(h2, 0.0)
    m = jnp.max(h2, axis=1, keepdims=True)
    lane = jax.lax.broadcasted_iota(jnp.int32, (TB, PAD), 1)
    idx = jnp.min(jnp.where(h2 >= m, lane, PAD), axis=1, keepdims=True)
    idx_ref[...] = idx


def _compute_idx(xf, W1, b1, W2, b2):
    N = xf.shape[0]
    w1p = jnp.zeros((1, PAD), jnp.float32).at[:, :BIN].set(W1)
    b1p = jnp.zeros((1, PAD), jnp.float32).at[:, :BIN].set(b1)
    w2p = jnp.zeros((PAD, PAD), jnp.float32).at[:BIN, :BIN].set(W2)
    b2p = jnp.zeros((1, PAD), jnp.float32).at[:, :BIN].set(b2)
    grid = N // TB
    idx = pl.pallas_call(
        _idx_body,
        grid=(grid,),
        in_specs=[
            pl.BlockSpec((TB, 1), lambda i: (i, 0)),
            pl.BlockSpec((1, PAD), lambda i: (0, 0)),
            pl.BlockSpec((1, PAD), lambda i: (0, 0)),
            pl.BlockSpec((PAD, PAD), lambda i: (0, 0)),
            pl.BlockSpec((1, PAD), lambda i: (0, 0)),
        ],
        out_specs=pl.BlockSpec((TB, 1), lambda i: (i, 0)),
        out_shape=jax.ShapeDtypeStruct((N, 1), jnp.int32),
    )(xf, w1p, b1p, w2p, b2p)
    return idx.reshape(N)


_INFO = plsc.get_sparse_core_info()
_NC, _NS = _INFO.num_cores, _INFO.num_subcores
_NW = _NC * _NS
_CHUNK = 128


def _make_sc_gather(N):
    b_per_w = N // _NW
    n_pairs = b_per_w // (2 * _CHUNK)
    mesh = plsc.VectorSubcoreMesh(core_axis_name="c", subcore_axis_name="s")

    @functools.partial(
        pl.kernel, mesh=mesh,
        out_type=jax.ShapeDtypeStruct((N, HID), jnp.float32),
        scratch_types=[
            pltpu.VMEM_SHARED((BIN, HID), jnp.float32),
            pltpu.VMEM((_CHUNK,), jnp.int32),
            pltpu.VMEM((_CHUNK,), jnp.int32),
            pltpu.VMEM((_CHUNK, HID), jnp.float32),
            pltpu.VMEM((_CHUNK, HID), jnp.float32),
            pltpu.SemaphoreType.DMA,
            pltpu.SemaphoreType.DMA,
        ],
        compiler_params=pltpu.CompilerParams(needs_layout_passes=False),
    )
    def sc_gather(emb_hbm, idx_hbm, out_hbm, emb_sh, idx0_v, idx1_v,
                  rows0_v, rows1_v, gsem, ssem):
        sid = lax.axis_index("s")
        wid = sid * _NC + lax.axis_index("c")
        base = wid * b_per_w

        @pl.when(sid == 0)
        def _():
            pltpu.sync_copy(emb_hbm, emb_sh)

        plsc.subcore_barrier()

        def fill(idx_v, rows_v, off):
            pltpu.sync_copy(idx_hbm.at[pl.ds(off, _CHUNK)], idx_v)
            pltpu.async_copy(emb_sh.at[idx_v], rows_v, gsem).wait()

        def pair(p, carry):
            off0 = base + (2 * p) * _CHUNK
            off1 = off0 + _CHUNK
            fill(idx0_v, rows0_v, off0)
            pltpu.async_copy(rows0_v, out_hbm.at[pl.ds(off0, _CHUNK)],
                             ssem).wait()
            fill(idx1_v, rows1_v, off1)
            pltpu.async_copy(rows1_v, out_hbm.at[pl.ds(off1, _CHUNK)],
                             ssem).wait()
            return carry

        lax.fori_loop(0, n_pairs, pair, 0)

    return sc_gather


def kernel(x, W1, b1, W2, b2, emb):
    B, L, _ = x.shape
    N = B * L
    xf = x.reshape(N, 1)
    idx = _compute_idx(xf, W1, b1, W2, b2)
    out = _make_sc_gather(N)(emb, idx)
    return out.reshape(B, L, HID)


# speedup vs baseline: 14.0570x; 1.0939x over previous
import functools

import jax
import jax.numpy as jnp
from jax import lax
from jax.experimental import pallas as pl
from jax.experimental.pallas import tpu as pltpu
from jax.experimental.pallas import tpu_sc as plsc

BIN = 100
PAD = 128
HID = 128
TB = 2048


def _idx_body(x_ref, w1_ref, b1_ref, w2_ref, b2_ref, idx_ref):
    xb = x_ref[...]
    h1 = jnp.maximum(xb * w1_ref[...] + b1_ref[...], 0.0)
    h2 = jax.lax.dot_general(
        h1, w2_ref[...], (((1,), (0,)), ((), ())),
        precision=jax.lax.Precision.DEFAULT,
        preferred_element_type=jnp.float32,
    ) + b2_ref[...]
    h2 = jnp.maximum(h2, 0.0)
    m = jnp.max(h2, axis=1, keepdims=True)
    lane = jax.lax.broadcasted_iota(jnp.int32, (TB, PAD), 1)
    idx = jnp.min(jnp.where(h2 >= m, lane, PAD), axis=1, keepdims=True)
    idx_ref[...] = idx


def _compute_idx(xf, W1, b1, W2, b2):
    N = xf.shape[0]
    w1p = jnp.zeros((1, PAD), jnp.float32).at[:, :BIN].set(W1)
    b1p = jnp.zeros((1, PAD), jnp.float32).at[:, :BIN].set(b1)
    w2p = jnp.zeros((PAD, PAD), jnp.float32).at[:BIN, :BIN].set(W2)
    b2p = jnp.zeros((1, PAD), jnp.float32).at[:, :BIN].set(b2)
    grid = N // TB
    idx = pl.pallas_call(
        _idx_body,
        grid=(grid,),
        in_specs=[
            pl.BlockSpec((TB, 1), lambda i: (i, 0)),
            pl.BlockSpec((1, PAD), lambda i: (0, 0)),
            pl.BlockSpec((1, PAD), lambda i: (0, 0)),
            pl.BlockSpec((PAD, PAD), lambda i: (0, 0)),
            pl.BlockSpec((1, PAD), lambda i: (0, 0)),
        ],
        out_specs=pl.BlockSpec((TB, 1), lambda i: (i, 0)),
        out_shape=jax.ShapeDtypeStruct((N, 1), jnp.int32),
    )(xf, w1p, b1p, w2p, b2p)
    return idx.reshape(N)


_INFO = plsc.get_sparse_core_info()
_NC, _NS = _INFO.num_cores, _INFO.num_subcores
_NW = _NC * _NS
_CHUNK = 128


def _make_sc_gather(N):
    b_per_w = N // _NW
    n_pairs = b_per_w // (2 * _CHUNK)
    mesh = plsc.VectorSubcoreMesh(core_axis_name="c", subcore_axis_name="s")

    @functools.partial(
        pl.kernel, mesh=mesh,
        out_type=jax.ShapeDtypeStruct((N, HID), jnp.float32),
        scratch_types=[
            pltpu.VMEM_SHARED((BIN, HID), jnp.float32),
            pltpu.VMEM((_CHUNK,), jnp.int32),
            pltpu.VMEM((_CHUNK,), jnp.int32),
            pltpu.VMEM((_CHUNK, HID), jnp.float32),
            pltpu.VMEM((_CHUNK, HID), jnp.float32),
            pltpu.SemaphoreType.DMA,
            pltpu.SemaphoreType.DMA,
            pltpu.SemaphoreType.DMA,
        ],
        compiler_params=pltpu.CompilerParams(needs_layout_passes=False),
    )
    def sc_gather(emb_hbm, idx_hbm, out_hbm, emb_sh, idx0_v, idx1_v,
                  rows0_v, rows1_v, gsem, ssem0, ssem1):
        sid = lax.axis_index("s")
        wid = sid * _NC + lax.axis_index("c")
        base = wid * b_per_w

        @pl.when(sid == 0)
        def _():
            pltpu.sync_copy(emb_hbm, emb_sh)

        plsc.subcore_barrier()

        def step(idx_v, rows_v, ssem, off):
            pltpu.sync_copy(idx_hbm.at[pl.ds(off, _CHUNK)], idx_v)
            pltpu.async_copy(emb_sh.at[idx_v], rows_v, gsem).wait()
            pltpu.async_copy(rows_v, out_hbm.at[pl.ds(off, _CHUNK)], ssem)

        def sdrain(rows_v, ssem):
            pltpu.make_async_copy(
                rows_v, out_hbm.at[pl.ds(0, _CHUNK)], ssem).wait()

        step(idx0_v, rows0_v, ssem0, base)
        step(idx1_v, rows1_v, ssem1, base + _CHUNK)

        def pair(p, carry):
            off0 = base + (2 * p) * _CHUNK
            sdrain(rows0_v, ssem0)
            step(idx0_v, rows0_v, ssem0, off0)
            sdrain(rows1_v, ssem1)
            step(idx1_v, rows1_v, ssem1, off0 + _CHUNK)
            return carry

        lax.fori_loop(1, n_pairs, pair, 0)
        sdrain(rows0_v, ssem0)
        sdrain(rows1_v, ssem1)

    return sc_gather


def kernel(x, W1, b1, W2, b2, emb):
    B, L, _ = x.shape
    N = B * L
    xf = x.reshape(N, 1)
    idx = _compute_idx(xf, W1, b1, W2, b2)
    out = _make_sc_gather(N)(emb, idx)
    return out.reshape(B, L, HID)


# speedup vs baseline: 14.6049x; 1.0390x over previous
import functools

import jax
import jax.numpy as jnp
from jax import lax
from jax.experimental import pallas as pl
from jax.experimental.pallas import tpu as pltpu
from jax.experimental.pallas import tpu_sc as plsc

BIN = 100
PAD = 128
HID = 128
TB = 2048


def _idx_body(x_ref, w1_ref, b1_ref, w2_ref, b2_ref, idx_ref):
    xb = x_ref[...]
    h1 = jnp.maximum(xb * w1_ref[...] + b1_ref[...], 0.0)
    h2 = jax.lax.dot_general(
        h1, w2_ref[...], (((1,), (0,)), ((), ())),
        precision=jax.lax.Precision.DEFAULT,
        preferred_element_type=jnp.float32,
    ) + b2_ref[...]
    h2 = jnp.maximum(h2, 0.0)
    m = jnp.max(h2, axis=1, keepdims=True)
    lane = jax.lax.broadcasted_iota(jnp.int32, (TB, PAD), 1)
    idx = jnp.min(jnp.where(h2 >= m, lane, PAD), axis=1, keepdims=True)
    idx_ref[...] = idx


def _compute_idx(xf, W1, b1, W2, b2):
    N = xf.shape[0]
    w1p = jnp.zeros((1, PAD), jnp.float32).at[:, :BIN].set(W1)
    b1p = jnp.zeros((1, PAD), jnp.float32).at[:, :BIN].set(b1)
    w2p = jnp.zeros((PAD, PAD), jnp.float32).at[:BIN, :BIN].set(W2)
    b2p = jnp.zeros((1, PAD), jnp.float32).at[:, :BIN].set(b2)
    grid = N // TB
    idx = pl.pallas_call(
        _idx_body,
        grid=(grid,),
        in_specs=[
            pl.BlockSpec((TB, 1), lambda i: (i, 0)),
            pl.BlockSpec((1, PAD), lambda i: (0, 0)),
            pl.BlockSpec((1, PAD), lambda i: (0, 0)),
            pl.BlockSpec((PAD, PAD), lambda i: (0, 0)),
            pl.BlockSpec((1, PAD), lambda i: (0, 0)),
        ],
        out_specs=pl.BlockSpec((TB, 1), lambda i: (i, 0)),
        out_shape=jax.ShapeDtypeStruct((N, 1), jnp.int32),
    )(xf, w1p, b1p, w2p, b2p)
    return idx.reshape(N)


_INFO = plsc.get_sparse_core_info()
_NC, _NS = _INFO.num_cores, _INFO.num_subcores
_NW = _NC * _NS
_CHUNK = 128


_NSLOT = 4


def _make_sc_gather(N):
    b_per_w = N // _NW
    n_rounds = b_per_w // (_NSLOT * _CHUNK)
    mesh = plsc.VectorSubcoreMesh(core_axis_name="c", subcore_axis_name="s")

    @functools.partial(
        pl.kernel, mesh=mesh,
        out_type=jax.ShapeDtypeStruct((N, HID), jnp.float32),
        scratch_types=(
            [pltpu.VMEM_SHARED((BIN, HID), jnp.float32)]
            + [pltpu.VMEM((_CHUNK,), jnp.int32)] * _NSLOT
            + [pltpu.VMEM((_CHUNK, HID), jnp.float32)] * _NSLOT
            + [pltpu.SemaphoreType.DMA] * (2 * _NSLOT)
        ),
        compiler_params=pltpu.CompilerParams(needs_layout_passes=False),
    )
    def sc_gather(emb_hbm, idx_hbm, out_hbm, emb_sh, *bufs):
        idx_v = bufs[:_NSLOT]
        rows_v = bufs[_NSLOT:2 * _NSLOT]
        gsem = bufs[2 * _NSLOT:3 * _NSLOT]
        ssem = bufs[3 * _NSLOT:4 * _NSLOT]
        sid = lax.axis_index("s")
        wid = sid * _NC + lax.axis_index("c")
        base = wid * b_per_w

        @pl.when(sid == 0)
        def _():
            pltpu.sync_copy(emb_hbm, emb_sh)

        plsc.subcore_barrier()

        def issue(s, off):
            pltpu.sync_copy(idx_hbm.at[pl.ds(off, _CHUNK)], idx_v[s])
            pltpu.async_copy(emb_sh.at[idx_v[s]], rows_v[s], gsem[s])

        def flush(s, off):
            pltpu.make_async_copy(emb_sh.at[idx_v[s]], rows_v[s],
                                  gsem[s]).wait()
            pltpu.async_copy(rows_v[s], out_hbm.at[pl.ds(off, _CHUNK)],
                             ssem[s])

        def sdrain(s):
            pltpu.make_async_copy(
                rows_v[s], out_hbm.at[pl.ds(0, _CHUNK)], ssem[s]).wait()

        for s in range(_NSLOT):
            issue(s, base + s * _CHUNK)

        def round_(p, carry):
            prev = base + (p - 1) * (_NSLOT * _CHUNK)
            cur = base + p * (_NSLOT * _CHUNK)
            for s in range(_NSLOT):
                flush(s, prev + s * _CHUNK)
            for s in range(_NSLOT):
                sdrain(s)
                issue(s, cur + s * _CHUNK)
            return carry

        lax.fori_loop(1, n_rounds, round_, 0)
        last = base + (n_rounds - 1) * (_NSLOT * _CHUNK)
        for s in range(_NSLOT):
            flush(s, last + s * _CHUNK)
        for s in range(_NSLOT):
            sdrain(s)

    return sc_gather


def kernel(x, W1, b1, W2, b2, emb):
    B, L, _ = x.shape
    N = B * L
    xf = x.reshape(N, 1)
    idx = _compute_idx(xf, W1, b1, W2, b2)
    out = _make_sc_gather(N)(emb, idx)
    return out.reshape(B, L, HID)


# speedup vs baseline: 30.8861x; 2.1148x over previous
import functools

import jax
import jax.numpy as jnp
from jax import lax
from jax.experimental import pallas as pl
from jax.experimental.pallas import tpu as pltpu
from jax.experimental.pallas import tpu_sc as plsc

BIN = 100
PAD = 128
HID = 128
TB = 2048


def _idx_body(x_ref, w1_ref, b1_ref, w2t_ref, b2_ref, pw_ref, idx_ref):
    xr = x_ref[0]
    h1 = jnp.maximum(w1_ref[...] * xr + b1_ref[...], 0.0)
    h2 = jax.lax.dot_general(
        w2t_ref[...], h1, (((1,), (0,)), ((), ())),
        precision=jax.lax.Precision.DEFAULT,
        preferred_element_type=jnp.float32,
    ) + b2_ref[...]
    h2 = jnp.maximum(h2, 0.0)
    m = jnp.max(h2, axis=0, keepdims=True)
    sel = (h2 >= m).astype(jnp.float32)
    rowval = jax.lax.dot_general(
        pw_ref[...], sel, (((1,), (0,)), ((), ())),
        precision=jax.lax.Precision.DEFAULT,
        preferred_element_type=jnp.float32,
    )
    bits = jax.lax.bitcast_convert_type(rowval, jnp.int32)
    idx = jnp.maximum(127 - (bits >> 23), 0)
    idx_ref[0] = idx


def _compute_idx(x2, W1, b1, W2, b2):
    G = x2.shape[0]
    w1c = jnp.zeros((PAD, 1), jnp.float32).at[:BIN, 0].set(W1[0])
    b1c = jnp.zeros((PAD, 1), jnp.float32).at[:BIN, 0].set(b1)
    w2t = jnp.zeros((PAD, PAD), jnp.float32).at[:BIN, :BIN].set(W2.T)
    b2c = jnp.zeros((PAD, 1), jnp.float32).at[:BIN, 0].set(b2)
    binr = jnp.arange(PAD, dtype=jnp.float32)
    pw = jnp.where(binr < BIN, jnp.exp2(-binr), 0.0).reshape(1, PAD)
    idx = pl.pallas_call(
        _idx_body,
        grid=(G,),
        in_specs=[
            pl.BlockSpec((1, 1, TB), lambda i: (i, 0, 0)),
            pl.BlockSpec((PAD, 1), lambda i: (0, 0)),
            pl.BlockSpec((PAD, 1), lambda i: (0, 0)),
            pl.BlockSpec((PAD, PAD), lambda i: (0, 0)),
            pl.BlockSpec((PAD, 1), lambda i: (0, 0)),
            pl.BlockSpec((1, PAD), lambda i: (0, 0)),
        ],
        out_specs=pl.BlockSpec((1, 1, TB), lambda i: (i, 0, 0)),
        out_shape=jax.ShapeDtypeStruct((G, 1, TB), jnp.int32),
    )(x2.reshape(G, 1, TB), w1c, b1c, w2t, b2c, pw)
    return idx.reshape(G * TB)


_INFO = plsc.get_sparse_core_info()
_NC, _NS = _INFO.num_cores, _INFO.num_subcores
_NW = _NC * _NS
_CHUNK = 128


_NSLOT = 4


def _make_sc_gather(N):
    b_per_w = N // _NW
    n_rounds = b_per_w // (_NSLOT * _CHUNK)
    mesh = plsc.VectorSubcoreMesh(core_axis_name="c", subcore_axis_name="s")

    @functools.partial(
        pl.kernel, mesh=mesh,
        out_type=jax.ShapeDtypeStruct((N, HID), jnp.float32),
        scratch_types=(
            [pltpu.VMEM_SHARED((BIN, HID), jnp.float32)]
            + [pltpu.VMEM((_CHUNK,), jnp.int32)] * _NSLOT
            + [pltpu.VMEM((_CHUNK, HID), jnp.float32)] * _NSLOT
            + [pltpu.SemaphoreType.DMA] * (2 * _NSLOT)
        ),
        compiler_params=pltpu.CompilerParams(needs_layout_passes=False),
    )
    def sc_gather(emb_hbm, idx_hbm, out_hbm, emb_sh, *bufs):
        idx_v = bufs[:_NSLOT]
        rows_v = bufs[_NSLOT:2 * _NSLOT]
        gsem = bufs[2 * _NSLOT:3 * _NSLOT]
        ssem = bufs[3 * _NSLOT:4 * _NSLOT]
        sid = lax.axis_index("s")
        wid = sid * _NC + lax.axis_index("c")
        base = wid * b_per_w

        @pl.when(sid == 0)
        def _():
            pltpu.sync_copy(emb_hbm, emb_sh)

        plsc.subcore_barrier()

        def issue(s, off):
            pltpu.sync_copy(idx_hbm.at[pl.ds(off, _CHUNK)], idx_v[s])
            pltpu.async_copy(emb_sh.at[idx_v[s]], rows_v[s], gsem[s])

        def flush(s, off):
            pltpu.make_async_copy(emb_sh.at[idx_v[s]], rows_v[s],
                                  gsem[s]).wait()
            pltpu.async_copy(rows_v[s], out_hbm.at[pl.ds(off, _CHUNK)],
                             ssem[s])

        def sdrain(s):
            pltpu.make_async_copy(
                rows_v[s], out_hbm.at[pl.ds(0, _CHUNK)], ssem[s]).wait()

        for s in range(_NSLOT):
            issue(s, base + s * _CHUNK)

        def round_(p, carry):
            prev = base + (p - 1) * (_NSLOT * _CHUNK)
            cur = base + p * (_NSLOT * _CHUNK)
            for s in range(_NSLOT):
                flush(s, prev + s * _CHUNK)
            for s in range(_NSLOT):
                sdrain(s)
                issue(s, cur + s * _CHUNK)
            return carry

        lax.fori_loop(1, n_rounds, round_, 0)
        last = base + (n_rounds - 1) * (_NSLOT * _CHUNK)
        for s in range(_NSLOT):
            flush(s, last + s * _CHUNK)
        for s in range(_NSLOT):
            sdrain(s)

    return sc_gather


def kernel(x, W1, b1, W2, b2, emb):
    B, L, _ = x.shape
    N = B * L
    x2 = x.reshape(N // TB, TB)
    idx = _compute_idx(x2, W1, b1, W2, b2)
    out = _make_sc_gather(N)(emb, idx)
    return out.reshape(B, L, HID)


# speedup vs baseline: 36.1133x; 1.1692x over previous
import functools

import jax
import jax.numpy as jnp
from jax import lax
from jax.experimental import pallas as pl
from jax.experimental.pallas import tpu as pltpu
from jax.experimental.pallas import tpu_sc as plsc

BIN = 100
PAD = 128
HID = 128
TB = 4096


def _idx_body(x_ref, w1_ref, b1_ref, w2t_ref, b2_ref, pw_ref, idx_ref):
    xr = x_ref[0]
    h1 = jnp.maximum(w1_ref[...] * xr + b1_ref[...], 0.0)
    h2 = jax.lax.dot_general(
        w2t_ref[...], h1, (((1,), (0,)), ((), ())),
        precision=jax.lax.Precision.DEFAULT,
        preferred_element_type=jnp.float32,
    ) + b2_ref[...]
    h2 = jnp.maximum(h2, 0.0)
    m = jnp.max(h2, axis=0, keepdims=True)
    sel = (h2 >= m).astype(jnp.float32)
    rowval = jax.lax.dot_general(
        pw_ref[...], sel, (((1,), (0,)), ((), ())),
        precision=jax.lax.Precision.DEFAULT,
        preferred_element_type=jnp.float32,
    )
    bits = jax.lax.bitcast_convert_type(rowval, jnp.int32)
    idx = jnp.maximum(127 - (bits >> 23), 0)
    idx_ref[0] = idx


def _compute_idx(x2, W1, b1, W2, b2):
    G = x2.shape[0]
    w1c = jnp.zeros((PAD, 1), jnp.float32).at[:BIN, 0].set(W1[0])
    b1c = jnp.zeros((PAD, 1), jnp.float32).at[:BIN, 0].set(b1)
    w2t = jnp.zeros((PAD, PAD), jnp.float32).at[:BIN, :BIN].set(W2.T)
    b2c = jnp.zeros((PAD, 1), jnp.float32).at[:BIN, 0].set(b2)
    binr = jnp.arange(PAD, dtype=jnp.float32)
    pw = jnp.where(binr < BIN, jnp.exp2(-binr), 0.0).reshape(1, PAD)
    idx = pl.pallas_call(
        _idx_body,
        grid=(G,),
        in_specs=[
            pl.BlockSpec((1, 1, TB), lambda i: (i, 0, 0)),
            pl.BlockSpec((PAD, 1), lambda i: (0, 0)),
            pl.BlockSpec((PAD, 1), lambda i: (0, 0)),
            pl.BlockSpec((PAD, PAD), lambda i: (0, 0)),
            pl.BlockSpec((PAD, 1), lambda i: (0, 0)),
            pl.BlockSpec((1, PAD), lambda i: (0, 0)),
        ],
        out_specs=pl.BlockSpec((1, 1, TB), lambda i: (i, 0, 0)),
        out_shape=jax.ShapeDtypeStruct((G, 1, TB), jnp.int32),
    )(x2.reshape(G, 1, TB), w1c, b1c, w2t, b2c, pw)
    return idx.reshape(G * TB)


_INFO = plsc.get_sparse_core_info()
_NC, _NS = _INFO.num_cores, _INFO.num_subcores
_NW = _NC * _NS
_CHUNK = 128


_NSLOT = 4


def _make_sc_gather(N):
    b_per_w = N // _NW
    n_rounds = b_per_w // (_NSLOT * _CHUNK)
    mesh = plsc.VectorSubcoreMesh(core_axis_name="c", subcore_axis_name="s")

    @functools.partial(
        pl.kernel, mesh=mesh,
        out_type=jax.ShapeDtypeStruct((N, HID), jnp.float32),
        scratch_types=(
            [pltpu.VMEM_SHARED((BIN, HID), jnp.float32)]
            + [pltpu.VMEM((_CHUNK,), jnp.int32)] * _NSLOT
            + [pltpu.VMEM((_CHUNK, HID), jnp.float32)] * _NSLOT
            + [pltpu.SemaphoreType.DMA] * (2 * _NSLOT)
        ),
        compiler_params=pltpu.CompilerParams(needs_layout_passes=False),
    )
    def sc_gather(emb_hbm, idx_hbm, out_hbm, emb_sh, *bufs):
        idx_v = bufs[:_NSLOT]
        rows_v = bufs[_NSLOT:2 * _NSLOT]
        gsem = bufs[2 * _NSLOT:3 * _NSLOT]
        ssem = bufs[3 * _NSLOT:4 * _NSLOT]
        sid = lax.axis_index("s")
        wid = sid * _NC + lax.axis_index("c")
        base = wid * b_per_w

        @pl.when(sid == 0)
        def _():
            pltpu.sync_copy(emb_hbm, emb_sh)

        plsc.subcore_barrier()

        def issue(s, off):
            pltpu.sync_copy(idx_hbm.at[pl.ds(off, _CHUNK)], idx_v[s])
            pltpu.async_copy(emb_sh.at[idx_v[s]], rows_v[s], gsem[s])

        def flush(s, off):
            pltpu.make_async_copy(emb_sh.at[idx_v[s]], rows_v[s],
                                  gsem[s]).wait()
            pltpu.async_copy(rows_v[s], out_hbm.at[pl.ds(off, _CHUNK)],
                             ssem[s])

        def sdrain(s):
            pltpu.make_async_copy(
                rows_v[s], out_hbm.at[pl.ds(0, _CHUNK)], ssem[s]).wait()

        for s in range(_NSLOT):
            issue(s, base + s * _CHUNK)

        def round_(p, carry):
            prev = base + (p - 1) * (_NSLOT * _CHUNK)
            cur = base + p * (_NSLOT * _CHUNK)
            for s in range(_NSLOT):
                flush(s, prev + s * _CHUNK)
            for s in range(_NSLOT):
                sdrain(s)
                issue(s, cur + s * _CHUNK)
            return carry

        lax.fori_loop(1, n_rounds, round_, 0)
        last = base + (n_rounds - 1) * (_NSLOT * _CHUNK)
        for s in range(_NSLOT):
            flush(s, last + s * _CHUNK)
        for s in range(_NSLOT):
            sdrain(s)

    return sc_gather


def kernel(x, W1, b1, W2, b2, emb):
    B, L, _ = x.shape
    N = B * L
    x2 = x.reshape(N // TB, TB)
    idx = _compute_idx(x2, W1, b1, W2, b2)
    out = _make_sc_gather(N)(emb, idx)
    return out.reshape(B, L, HID)
